# Initial kernel scaffold; baseline (speedup 1.0000x reference)
import jax
import jax.numpy as jnp
from jax.experimental import pallas as pl


def kernel(xyz1, xyz2, features1, features2):
    raise NotImplementedError("write your pallas kernel here")


# speedup vs baseline: 20.3791x; 20.3791x over previous
import functools

import jax
import jax.numpy as jnp
from jax import lax
from jax.experimental import pallas as pl


def _body(x1_ref, x2t_ref, f1_ref, f2_ref, out_ref, *, n2, c2, tile):
    x1 = x1_ref[0]
    x2 = x2t_ref[0]

    e0 = x2[:, 0:1] - x1[0:1, :]
    e1 = x2[:, 1:2] - x1[1:2, :]
    e2 = x2[:, 2:3] - x1[2:3, :]
    d = (e0 * e0 + e1 * e1) + e2 * e2

    iota = lax.broadcasted_iota(jnp.int32, (n2, tile), 0)
    dists, idxs = [], []
    for _ in range(3):
        m = jnp.min(d, axis=0, keepdims=True)
        a = jnp.min(jnp.where(d == m, iota, n2), axis=0, keepdims=True)
        d = jnp.where(iota == a, jnp.inf, d)
        dists.append(m)
        idxs.append(a)

    rs = [1.0 / jnp.maximum(m, 1e-10) for m in dists]
    norm = (rs[0] + rs[1]) + rs[2]

    w = jnp.zeros((n2, tile), jnp.float32)
    for k in range(3):
        w = w + jnp.where(iota == idxs[k], rs[k] / norm, 0.0)

    interp = lax.dot_general(
        f2_ref[0], w, (((1,), (0,)), ((), ())),
        preferred_element_type=jnp.float32,
        precision=lax.Precision.HIGHEST,
    )
    out_ref[0, :c2, :] = interp
    out_ref[0, c2:, :] = f1_ref[0]


def kernel(xyz1, xyz2, features1, features2):
    b, _, n1 = xyz1.shape
    n2 = xyz2.shape[2]
    c1 = features1.shape[1]
    c2 = features2.shape[1]
    tile = min(256, n1)

    xyz2t = jnp.transpose(xyz2, (0, 2, 1))

    grid = (b, n1 // tile)
    return pl.pallas_call(
        functools.partial(_body, n2=n2, c2=c2, tile=tile),
        grid=grid,
        in_specs=[
            pl.BlockSpec((1, 3, tile), lambda ib, it: (ib, 0, it)),
            pl.BlockSpec((1, n2, 3), lambda ib, it: (ib, 0, 0)),
            pl.BlockSpec((1, c1, tile), lambda ib, it: (ib, 0, it)),
            pl.BlockSpec((1, c2, n2), lambda ib, it: (ib, 0, 0)),
        ],
        out_specs=pl.BlockSpec((1, c1 + c2, tile), lambda ib, it: (ib, 0, it)),
        out_shape=jax.ShapeDtypeStruct((b, c1 + c2, n1), jnp.float32),
    )(xyz1, xyz2t, features1, features2)


# speedup vs baseline: 38.1739x; 1.8732x over previous
import functools

import jax
import jax.numpy as jnp
from jax import lax
from jax.experimental import pallas as pl


def _body(x1_ref, x2t_ref, f1_ref, f2_ref, out_ref, *, n2, c2, tile):
    x1 = x1_ref[0]
    x2 = x2t_ref[0]

    e0 = x2[:, 0:1] - x1[0:1, :]
    e1 = x2[:, 1:2] - x1[1:2, :]
    e2 = x2[:, 2:3] - x1[2:3, :]
    d = (e0 * e0 + e1 * e1) + e2 * e2

    iota = lax.broadcasted_iota(jnp.int32, (n2, tile), 0)
    dists, idxs = [], []
    for _ in range(3):
        m = jnp.min(d, axis=0, keepdims=True)
        a = jnp.min(jnp.where(d == m, iota, n2), axis=0, keepdims=True)
        d = jnp.where(iota == a, jnp.inf, d)
        dists.append(m)
        idxs.append(a)

    rs = [1.0 / jnp.maximum(m, 1e-10) for m in dists]
    norm = (rs[0] + rs[1]) + rs[2]

    w = jnp.zeros((n2, tile), jnp.float32)
    for k in range(3):
        w = w + jnp.where(iota == idxs[k], rs[k] / norm, 0.0)

    interp = lax.dot_general(
        f2_ref[0], w, (((1,), (0,)), ((), ())),
        preferred_element_type=jnp.float32,
        precision=lax.Precision.DEFAULT,
    )
    out_ref[0, :c2, :] = interp
    out_ref[0, c2:, :] = f1_ref[0]


def kernel(xyz1, xyz2, features1, features2):
    b, _, n1 = xyz1.shape
    n2 = xyz2.shape[2]
    c1 = features1.shape[1]
    c2 = features2.shape[1]
    tile = min(512, n1)

    xyz2t = jnp.transpose(xyz2, (0, 2, 1))

    grid = (b, n1 // tile)
    return pl.pallas_call(
        functools.partial(_body, n2=n2, c2=c2, tile=tile),
        grid=grid,
        in_specs=[
            pl.BlockSpec((1, 3, tile), lambda ib, it: (ib, 0, it)),
            pl.BlockSpec((1, n2, 3), lambda ib, it: (ib, 0, 0)),
            pl.BlockSpec((1, c1, tile), lambda ib, it: (ib, 0, it)),
            pl.BlockSpec((1, c2, n2), lambda ib, it: (ib, 0, 0)),
        ],
        out_specs=pl.BlockSpec((1, c1 + c2, tile), lambda ib, it: (ib, 0, it)),
        out_shape=jax.ShapeDtypeStruct((b, c1 + c2, n1), jnp.float32),
    )(xyz1, xyz2t, features1, features2)


# speedup vs baseline: 45.2980x; 1.1866x over previous
import functools

import jax
import jax.numpy as jnp
from jax import lax
from jax.experimental import pallas as pl


def _body(x1_ref, x2t_ref, f1_ref, f2_ref, out_ref, *, n2, c2, tile):
    x1 = x1_ref[0]
    x2 = x2t_ref[0]

    e0 = x2[:, 0:1] - x1[0:1, :]
    e1 = x2[:, 1:2] - x1[1:2, :]
    e2 = x2[:, 2:3] - x1[2:3, :]
    d = (e0 * e0 + e1 * e1) + e2 * e2

    fio = lax.broadcasted_iota(jnp.int32, (n2, tile), 0).astype(jnp.float32)
    dists, eqms = [], []
    for _ in range(3):
        m = jnp.min(d, axis=0, keepdims=True)
        af = jnp.min(jnp.where(d == m, fio, 1e9), axis=0, keepdims=True)
        eqm = fio == af
        d = jnp.where(eqm, jnp.inf, d)
        dists.append(m)
        eqms.append(eqm)

    rs = [1.0 / jnp.maximum(m, 1e-10) for m in dists]
    norm = (rs[0] + rs[1]) + rs[2]

    w = jnp.where(
        eqms[0], rs[0] / norm,
        jnp.where(eqms[1], rs[1] / norm,
                  jnp.where(eqms[2], rs[2] / norm, 0.0)))

    interp = lax.dot_general(
        f2_ref[0], w, (((1,), (0,)), ((), ())),
        preferred_element_type=jnp.float32,
        precision=lax.Precision.DEFAULT,
    )
    out_ref[0, :c2, :] = interp
    out_ref[0, c2:, :] = f1_ref[0]


def kernel(xyz1, xyz2, features1, features2):
    b, _, n1 = xyz1.shape
    n2 = xyz2.shape[2]
    c1 = features1.shape[1]
    c2 = features2.shape[1]
    tile = min(512, n1)

    xyz2t = jnp.transpose(xyz2, (0, 2, 1))

    grid = (b, n1 // tile)
    return pl.pallas_call(
        functools.partial(_body, n2=n2, c2=c2, tile=tile),
        grid=grid,
        in_specs=[
            pl.BlockSpec((1, 3, tile), lambda ib, it: (ib, 0, it)),
            pl.BlockSpec((1, n2, 3), lambda ib, it: (ib, 0, 0)),
            pl.BlockSpec((1, c1, tile), lambda ib, it: (ib, 0, it)),
            pl.BlockSpec((1, c2, n2), lambda ib, it: (ib, 0, 0)),
        ],
        out_specs=pl.BlockSpec((1, c1 + c2, tile), lambda ib, it: (ib, 0, it)),
        out_shape=jax.ShapeDtypeStruct((b, c1 + c2, n1), jnp.float32),
    )(xyz1, xyz2t, features1, features2)
